# baseline (speedup 1.0000x reference)
import jax
import jax.numpy as jnp
from jax import lax
from jax.experimental import pallas as pl
from jax.experimental.pallas import tpu as pltpu
from jax.experimental.pallas import tpu_sc as plsc

EMBED_DIM = 300
N_EDGES = 160000

NQ = N_EDGES // 4

NC = 2
NS = 16
NW = NC * NS
PER_W = NQ // NW
QCHUNK = 25
NCHUNK = PER_W // QCHUNK


NBUF = 3


def _sc_body(stable_hbm, sidx_hbm, out_hbm, t_sh, sidx_v, r0, r1, r2,
             g0, g1, g2, s0, s1, s2):
    wid = lax.axis_index("s") * NC + lax.axis_index("c")
    base = wid * PER_W
    @pl.when(lax.axis_index("s") == 0)
    def _():
        pltpu.sync_copy(stable_hbm, t_sh)
    pltpu.sync_copy(sidx_hbm.at[wid], sidx_v)
    plsc.subcore_barrier()

    rows = (r0, r1, r2)
    gsem = (g0, g1, g2)
    ssem = (s0, s1, s2)
    stores = [None] * NBUF

    pltpu.async_copy(t_sh.at[sidx_v.at[0]], rows[0], gsem[0])
    for c in range(NCHUNK):
        b = c % NBUF
        if c + 1 < NCHUNK:
            nb = (c + 1) % NBUF
            if stores[nb] is not None:
                stores[nb].wait()
            pltpu.async_copy(t_sh.at[sidx_v.at[c + 1]], rows[nb], gsem[nb])
        pltpu.make_async_copy(t_sh.at[sidx_v.at[c]], rows[b], gsem[b]).wait()
        stores[b] = pltpu.async_copy(
            rows[b], out_hbm.at[pl.ds(base + c * QCHUNK, QCHUNK)], ssem[b]
        )
    for b in range(min(NBUF, NCHUNK)):
        if stores[b] is not None:
            stores[b].wait()


def kernel(tensor, table):
    table = table.astype(jnp.float32)
    idx = tensor.astype(jnp.int32)

    digits = jax.lax.broadcasted_iota(jnp.int32, (256, 4), 0)
    shifts = jnp.array([6, 4, 2, 0], jnp.int32)
    combo = (digits >> shifts[None, :]) & 3
    stable = jnp.take(table, combo.reshape(-1), axis=0).reshape(256, 4 * EMBED_DIM)

    q = idx.reshape(NQ, 4)
    sidx = ((q[:, 0] * 4 + q[:, 1]) * 4 + q[:, 2]) * 4 + q[:, 3]
    sidx = sidx.reshape(NW, NCHUNK, QCHUNK)

    mesh = plsc.VectorSubcoreMesh(
        core_axis_name="c", subcore_axis_name="s", num_cores=NC, num_subcores=NS
    )
    run = pl.kernel(
        _sc_body,
        out_type=jax.ShapeDtypeStruct((NQ, 4 * EMBED_DIM), jnp.float32),
        mesh=mesh,
        scratch_types=[
            pltpu.VMEM_SHARED((256, 4 * EMBED_DIM), jnp.float32),
            pltpu.VMEM((NCHUNK, QCHUNK), jnp.int32),
            pltpu.VMEM((QCHUNK, 4 * EMBED_DIM), jnp.float32),
            pltpu.VMEM((QCHUNK, 4 * EMBED_DIM), jnp.float32),
            pltpu.VMEM((QCHUNK, 4 * EMBED_DIM), jnp.float32),
            pltpu.SemaphoreType.DMA,
            pltpu.SemaphoreType.DMA,
            pltpu.SemaphoreType.DMA,
            pltpu.SemaphoreType.DMA,
            pltpu.SemaphoreType.DMA,
            pltpu.SemaphoreType.DMA,
        ],
        compiler_params=pltpu.CompilerParams(use_tc_tiling_on_sc=False),
    )
    out = run(stable, sidx)
    return out.reshape(N_EDGES, EMBED_DIM)


# speedup vs baseline: 1.0147x; 1.0147x over previous
import jax
import jax.numpy as jnp
from jax import lax
from jax.experimental import pallas as pl
from jax.experimental.pallas import tpu as pltpu
from jax.experimental.pallas import tpu_sc as plsc

EMBED_DIM = 300
N_EDGES = 160000

NQ = N_EDGES // 4

NC = 2
NS = 16
NW = NC * NS
PER_W = NQ // NW
QCHUNK = 25
NCHUNK = PER_W // QCHUNK


NBUF = 2


def _sc_body(stable_hbm, sidx_hbm, out_hbm, t_sh, sidx_v, r0, r1,
             g0, g1, s0, s1):
    wid = lax.axis_index("s") * NC + lax.axis_index("c")
    base = wid * PER_W
    @pl.when(lax.axis_index("s") == 0)
    def _():
        pltpu.sync_copy(stable_hbm, t_sh)
    pltpu.sync_copy(sidx_hbm.at[wid], sidx_v)
    plsc.subcore_barrier()

    rows = (r0, r1)
    gsem = (g0, g1)
    ssem = (s0, s1)
    stores = [None] * NBUF

    pltpu.async_copy(t_sh.at[sidx_v.at[0]], rows[0], gsem[0])
    for c in range(NCHUNK):
        b = c % NBUF
        if c + 1 < NCHUNK:
            nb = (c + 1) % NBUF
            if stores[nb] is not None:
                stores[nb].wait()
            pltpu.async_copy(t_sh.at[sidx_v.at[c + 1]], rows[nb], gsem[nb])
        pltpu.make_async_copy(t_sh.at[sidx_v.at[c]], rows[b], gsem[b]).wait()
        stores[b] = pltpu.async_copy(
            rows[b], out_hbm.at[pl.ds(base + c * QCHUNK, QCHUNK)], ssem[b]
        )
    for b in range(min(NBUF, NCHUNK)):
        if stores[b] is not None:
            stores[b].wait()


def kernel(tensor, table):
    table = table.astype(jnp.float32)
    idx = tensor.astype(jnp.int32)

    digits = jax.lax.broadcasted_iota(jnp.int32, (256, 4), 0)
    shifts = jnp.array([6, 4, 2, 0], jnp.int32)
    combo = (digits >> shifts[None, :]) & 3
    stable = jnp.take(table, combo.reshape(-1), axis=0).reshape(256, 4 * EMBED_DIM)

    q = idx.reshape(NQ, 4)
    sidx = ((q[:, 0] * 4 + q[:, 1]) * 4 + q[:, 2]) * 4 + q[:, 3]
    sidx = sidx.reshape(NW, NCHUNK, QCHUNK)

    mesh = plsc.VectorSubcoreMesh(
        core_axis_name="c", subcore_axis_name="s", num_cores=NC, num_subcores=NS
    )
    run = pl.kernel(
        _sc_body,
        out_type=jax.ShapeDtypeStruct((NQ, 4 * EMBED_DIM), jnp.float32),
        mesh=mesh,
        scratch_types=[
            pltpu.VMEM_SHARED((256, 4 * EMBED_DIM), jnp.float32),
            pltpu.VMEM((NCHUNK, QCHUNK), jnp.int32),
            pltpu.VMEM((QCHUNK, 4 * EMBED_DIM), jnp.float32),
            pltpu.VMEM((QCHUNK, 4 * EMBED_DIM), jnp.float32),
            pltpu.SemaphoreType.DMA,
            pltpu.SemaphoreType.DMA,
            pltpu.SemaphoreType.DMA,
            pltpu.SemaphoreType.DMA,
        ],
        compiler_params=pltpu.CompilerParams(use_tc_tiling_on_sc=False),
    )
    out = run(stable, sidx)
    return out.reshape(N_EDGES, EMBED_DIM)


# speedup vs baseline: 1.8912x; 1.8639x over previous
import jax
import jax.numpy as jnp
from jax import lax
from jax.experimental import pallas as pl
from jax.experimental.pallas import tpu as pltpu
from jax.experimental.pallas import tpu_sc as plsc

EMBED_DIM = 300
N_EDGES = 160000

DP = 384
NT = DP // 128
NG = N_EDGES // 8

NC = 2
NS = 16
NW = NC * NS
PER_W = NG // NW * 8 * NT
CHUNK = 120
NCHUNK = PER_W // CHUNK


def _sc_body(ctab_hbm, cidx_hbm, out_hbm, ctab_v, cidx_v, r0, r1, g0, g1, s0, s1):
    wid = lax.axis_index("s") * NC + lax.axis_index("c")
    base = wid * PER_W
    @pl.when(lax.axis_index("s") == 0)
    def _():
        pltpu.sync_copy(ctab_hbm, ctab_v)
    pltpu.sync_copy(cidx_hbm.at[wid], cidx_v)
    plsc.subcore_barrier()

    rows = (r0, r1)
    gsem = (g0, g1)
    ssem = (s0, s1)
    stores = [None, None]

    pltpu.async_copy(ctab_v.at[cidx_v.at[0]], rows[0], gsem[0])
    for c in range(NCHUNK):
        b = c % 2
        if c + 1 < NCHUNK:
            nb = (c + 1) % 2
            if stores[nb] is not None:
                stores[nb].wait()
            pltpu.async_copy(ctab_v.at[cidx_v.at[c + 1]], rows[nb], gsem[nb])
        pltpu.make_async_copy(ctab_v.at[cidx_v.at[c]], rows[b], gsem[b]).wait()
        stores[b] = pltpu.async_copy(
            rows[b], out_hbm.at[pl.ds(base + c * CHUNK, CHUNK)], ssem[b]
        )
    stores[0].wait()
    stores[1].wait()


def kernel(tensor, table):
    table = table.astype(jnp.float32)
    idx = tensor.astype(jnp.int32)

    ctab = jnp.pad(table, ((0, 0), (0, DP - EMBED_DIM))).reshape(4 * NT, 128)

    v8 = idx.reshape(NG, 1, 8)
    cidx = 3 * v8 + jax.lax.broadcasted_iota(jnp.int32, (NG, NT, 8), 1)
    cidx = cidx.reshape(NW, NCHUNK, CHUNK)

    mesh = plsc.VectorSubcoreMesh(
        core_axis_name="c", subcore_axis_name="s", num_cores=NC, num_subcores=NS
    )
    run = pl.kernel(
        _sc_body,
        out_type=jax.ShapeDtypeStruct((NG * NT * 8, 128), jnp.float32),
        mesh=mesh,
        scratch_types=[
            pltpu.VMEM_SHARED((4 * NT, 128), jnp.float32),
            pltpu.VMEM((NCHUNK, CHUNK), jnp.int32),
            pltpu.VMEM((CHUNK, 128), jnp.float32),
            pltpu.VMEM((CHUNK, 128), jnp.float32),
            pltpu.SemaphoreType.DMA,
            pltpu.SemaphoreType.DMA,
            pltpu.SemaphoreType.DMA,
            pltpu.SemaphoreType.DMA,
        ],
        compiler_params=pltpu.CompilerParams(use_tc_tiling_on_sc=False),
    )
    out2 = run(ctab, cidx)
    out = out2.reshape(NG, NT, 8, 128).transpose(0, 2, 1, 3)
    return out.reshape(N_EDGES, DP)[:, :EMBED_DIM]
